# baseline (speedup 1.0000x reference)
import jax
import jax.numpy as jnp
from jax import lax
from jax.experimental import pallas as pl
from jax.experimental.pallas import tpu as pltpu
from jax.experimental.pallas import tpu_sc as plsc

VOCAB = 1000000
DIM = 64
B = 16384

NUM_CORES = 2
NUM_SUBCORES = 16
LANES = 16
NW = NUM_CORES * NUM_SUBCORES
BPW = B // NW

VTAIL = 999936
WIN = 256
NWIN = VTAIL // WIN
WPW_LO = NWIN // NW
WREM = NWIN - WPW_LO * NW
RING = 4
NG = 2
CAP = 1024
NEDGE = 3920
DUMP = B
IROWS = B + 16


def _sweep_body(st_hbm, pt_hbm, sc_hbm, pc_hbm, bt_hbm, bc_hbm, table_hbm,
                u2_hbm, v2_hbm,
                btv, bcv, sval, pval, cval, qval,
                win0, win1, win2, win3, rows, posb,
                semw0, semw1, semw2, semw3, sems):
    wid = lax.axis_index("s") * NUM_CORES + lax.axis_index("c")
    j0 = wid * WPW_LO + jnp.minimum(wid, WREM)
    nwin = jnp.where(wid < WREM, WPW_LO + 1, WPW_LO)

    pltpu.sync_copy(bt_hbm, btv)
    pltpu.sync_copy(bc_hbm, bcv)

    wins = [win0, win1, win2, win3]
    semw = [semw0, semw1, semw2, semw3]

    def stage(bv, v_hbm, p_hbm, vdst, pdst):
        s16 = bv[pl.ds(j0, LANES)]
        off = jnp.minimum((s16[0] // 8) * 8, B - CAP)
        off = pl.multiple_of(off, 8)
        pltpu.sync_copy(v_hbm.at[pl.ds(off, CAP)], vdst)
        pltpu.sync_copy(p_hbm.at[pl.ds(off, CAP)], pdst)
        return off

    toff = stage(btv, st_hbm, pt_hbm, sval, pval)
    coff = stage(bcv, sc_hbm, pc_hbm, cval, qval)

    lanes = lax.iota(jnp.int32, LANES)

    def fire_load(j, r):
        @pl.when(j < j0 + nwin)
        def _():
            wbase = pl.multiple_of(j * WIN, 128)
            pltpu.async_copy(
                table_hbm.at[pl.ds(0, DIM), pl.ds(wbase, WIN)],
                wins[r], semw[r])

    for r in range(RING):
        fire_load(j0 + r, r)
    for sl in range(2 * NG):
        posb[sl, :] = jnp.full((LANES,), DUMP + sl, jnp.int32)
        pltpu.async_copy(rows.at[sl], u2_hbm.at[posb.at[sl]], sems)

    def do_window(j, r):
        for sl in range(2 * NG):
            pltpu.make_async_copy(
                u2_hbm.at[pl.ds(0, LANES), pl.ds(0, 2 * DIM)],
                rows.at[sl], sems).wait()
        pltpu.make_async_copy(
            table_hbm.at[pl.ds(0, DIM), pl.ds(0, WIN)],
            wins[r], semw[r]).wait()
        wbase = j * WIN

        def one_table(bv, vals, poss, off, dst_hbm, tsl):
            e16 = bv[pl.ds(j, LANES)]
            s = e16[0]
            e = e16[1]
            for g in range(NG):
                sl = tsl * NG + g
                posb[sl, :] = jnp.full((LANES,), DUMP + sl, jnp.int32)
                i = s + g * LANES

                @pl.when(i < e)
                def _():
                    li = i - off
                    sv16 = vals[pl.ds(li, LANES)]
                    sp16 = poss[pl.ds(li, LANES)]
                    msk = (lanes + i) < e
                    cols = jnp.clip(sv16 - wbase, 0, WIN - 1)
                    posb[sl, :] = jnp.where(msk, sp16, DUMP + sl)
                    for d in range(DIM):
                        gv = plsc.load_gather(
                            wins[r], [jnp.full((LANES,), d, jnp.int32), cols])
                        plsc.store_scatter(
                            rows.at[sl],
                            [lanes, jnp.full((LANES,), d, jnp.int32)], gv)

                pltpu.async_copy(rows.at[sl], dst_hbm.at[posb.at[sl]], sems)

        one_table(btv, sval, pval, toff, u2_hbm, 0)
        one_table(bcv, cval, qval, coff, v2_hbm, 1)
        fire_load(j + RING, r)

    nsuper = (WPW_LO + 1 + RING - 1) // RING

    def super_iter(q, carry):
        for r in range(RING):
            j = j0 + q * RING + r

            @pl.when(j < j0 + nwin)
            def _():
                do_window(j, r)
        return carry

    lax.fori_loop(0, nsuper, super_iter, 0)

    for sl in range(2 * NG):
        pltpu.make_async_copy(
            u2_hbm.at[pl.ds(0, LANES), pl.ds(0, 2 * DIM)],
            rows.at[sl], sems).wait()


def _dot_body(t_hbm, c_hbm, u2_hbm, v2_hbm, aux_hbm, out_hbm,
              tv, cv, ub, vb, auxv, out_v, sem):
    wid = lax.axis_index("s") * NUM_CORES + lax.axis_index("c")
    base = wid * BPW

    pltpu.sync_copy(t_hbm.at[pl.ds(base, BPW)], tv)
    pltpu.sync_copy(c_hbm.at[pl.ds(base, BPW)], cv)
    pltpu.sync_copy(aux_hbm, auxv)

    lanes = lax.iota(jnp.int32, LANES)
    NCH = BPW // 128

    for ch in range(NCH):
        row0 = pl.multiple_of(base + ch * 128, 8)
        pltpu.sync_copy(u2_hbm.at[pl.ds(row0, 128), pl.ds(0, 2 * DIM)], ub)
        pltpu.sync_copy(v2_hbm.at[pl.ds(row0, 128), pl.ds(0, 2 * DIM)], vb)

        def group(g, carry):
            i0 = ch * 128 + g * LANES
            t16 = tv[pl.ds(i0, LANES)]
            c16 = cv[pl.ds(i0, LANES)]
            tm = t16 >= VTAIL
            cm = c16 >= VTAIL
            ta = jnp.clip(t16 - VTAIL, 0, DIM - 1)
            ca = jnp.clip(c16 - VTAIL, 0, DIM - 1)
            rloc = lanes + g * LANES
            acc = jnp.zeros((LANES,), jnp.float32)
            for d in range(DIM):
                dsp = jnp.full((LANES,), d, jnp.int32)
                uu = plsc.load_gather(ub, [rloc, dsp])
                vv = plsc.load_gather(vb, [rloc, dsp])
                au = plsc.load_gather(auxv, [ta, dsp])
                av = plsc.load_gather(auxv, [ca, dsp])
                u = jnp.where(tm, au, uu)
                v = jnp.where(cm, av, vv)
                acc = acc + u * v
            out_v[pl.ds(i0, LANES)] = acc
            return carry

        lax.fori_loop(0, 8, group, 0)

    pltpu.sync_copy(out_v, out_hbm.at[pl.ds(base, BPW)])


@jax.jit
def _skipgram(t32, c32, table_t, aux):
    pt = jnp.argsort(t32).astype(jnp.int32)
    st = t32[pt]
    pc = jnp.argsort(c32).astype(jnp.int32)
    sc_ = c32[pc]
    edges = jnp.minimum(jnp.arange(NEDGE, dtype=jnp.int32) * WIN, VOCAB)
    bt = jnp.searchsorted(st, edges).astype(jnp.int32)
    bc = jnp.searchsorted(sc_, edges).astype(jnp.int32)

    mesh = plsc.VectorSubcoreMesh(core_axis_name="c", subcore_axis_name="s")
    u2, v2 = pl.kernel(
        _sweep_body,
        out_type=(jax.ShapeDtypeStruct((IROWS, 2 * DIM), jnp.float32),
                  jax.ShapeDtypeStruct((IROWS, 2 * DIM), jnp.float32)),
        mesh=mesh,
        scratch_types=[
            pltpu.VMEM((NEDGE,), jnp.int32),
            pltpu.VMEM((NEDGE,), jnp.int32),
            pltpu.VMEM((CAP,), jnp.int32),
            pltpu.VMEM((CAP,), jnp.int32),
            pltpu.VMEM((CAP,), jnp.int32),
            pltpu.VMEM((CAP,), jnp.int32),
            pltpu.VMEM((DIM, WIN), jnp.float32),
            pltpu.VMEM((DIM, WIN), jnp.float32),
            pltpu.VMEM((DIM, WIN), jnp.float32),
            pltpu.VMEM((DIM, WIN), jnp.float32),
            pltpu.VMEM((2 * NG, LANES, 2 * DIM), jnp.float32),
            pltpu.VMEM((2 * NG, LANES), jnp.int32),
            pltpu.SemaphoreType.DMA,
            pltpu.SemaphoreType.DMA,
            pltpu.SemaphoreType.DMA,
            pltpu.SemaphoreType.DMA,
            pltpu.SemaphoreType.DMA,
        ],
        compiler_params=pltpu.CompilerParams(needs_layout_passes=False),
    )(st, pt, sc_, pc, bt, bc, table_t)

    return pl.kernel(
        _dot_body,
        out_type=jax.ShapeDtypeStruct((B,), jnp.float32),
        mesh=mesh,
        scratch_types=[
            pltpu.VMEM((BPW,), jnp.int32),
            pltpu.VMEM((BPW,), jnp.int32),
            pltpu.VMEM((128, 2 * DIM), jnp.float32),
            pltpu.VMEM((128, 2 * DIM), jnp.float32),
            pltpu.VMEM((DIM, 2 * DIM), jnp.float32),
            pltpu.VMEM((BPW,), jnp.float32),
            pltpu.SemaphoreType.DMA,
        ],
        compiler_params=pltpu.CompilerParams(needs_layout_passes=False),
    )(t32, c32, u2, v2, aux)


def kernel(target, context, embedding_weights):
    t32 = target.astype(jnp.int32)
    c32 = context.astype(jnp.int32)
    aux = jnp.pad(embedding_weights[VTAIL:], ((0, 0), (0, DIM)))
    return _skipgram(t32, c32, embedding_weights.T, aux)


# speedup vs baseline: 1.7074x; 1.7074x over previous
import jax
import jax.numpy as jnp
from jax import lax
from jax.experimental import pallas as pl
from jax.experimental.pallas import tpu as pltpu
from jax.experimental.pallas import tpu_sc as plsc

VOCAB = 1000000
DIM = 64
B = 16384

NUM_CORES = 2
NUM_SUBCORES = 16
LANES = 16
NW = NUM_CORES * NUM_SUBCORES
BPW = B // NW

VTAIL = 999936
WIN = 512
NWIN = VTAIL // WIN
WPW_LO = NWIN // NW
WREM = NWIN - WPW_LO * NW
RING = 2
NG = 3
CAP = 1024
NEDGE = 1968
DUMP = B
IROWS = B + 16


def _sweep_body(st_hbm, pt_hbm, sc_hbm, pc_hbm, bt_hbm, bc_hbm, table_hbm,
                u2_hbm, v2_hbm,
                btv, bcv, sval, pval, cval, qval,
                win0, win1, rows, posb,
                semw0, semw1, sems):
    wid = lax.axis_index("s") * NUM_CORES + lax.axis_index("c")
    j0 = wid * WPW_LO + jnp.minimum(wid, WREM)
    nwin = jnp.where(wid < WREM, WPW_LO + 1, WPW_LO)

    pltpu.sync_copy(bt_hbm, btv)
    pltpu.sync_copy(bc_hbm, bcv)

    wins = [win0, win1]
    semw = [semw0, semw1]

    def stage(bv, v_hbm, p_hbm, vdst, pdst):
        s16 = bv[pl.ds(j0, LANES)]
        off = jnp.minimum((s16[0] // 8) * 8, B - CAP)
        off = pl.multiple_of(off, 8)
        pltpu.sync_copy(v_hbm.at[pl.ds(off, CAP)], vdst)
        pltpu.sync_copy(p_hbm.at[pl.ds(off, CAP)], pdst)
        return off

    toff = stage(btv, st_hbm, pt_hbm, sval, pval)
    coff = stage(bcv, sc_hbm, pc_hbm, cval, qval)

    lanes = lax.iota(jnp.int32, LANES)

    def fire_load(j, r):
        @pl.when(j < j0 + nwin)
        def _():
            wbase = pl.multiple_of(j * WIN, 128)
            for tr in range(DIM // 8):
                pltpu.async_copy(
                    table_hbm.at[pl.ds(tr * 8, 8), pl.ds(wbase, WIN)],
                    wins[r].at[pl.ds(tr * 8, 8), pl.ds(0, WIN)], semw[r])

    for r in range(RING):
        fire_load(j0 + r, r)
    for sl in range(2 * NG):
        posb[sl, :] = jnp.full((LANES,), DUMP + sl, jnp.int32)
        pltpu.async_copy(rows.at[sl], u2_hbm.at[posb.at[sl]], sems)

    def do_window(j, r):
        for sl in range(2 * NG):
            pltpu.make_async_copy(
                u2_hbm.at[pl.ds(0, LANES), pl.ds(0, 2 * DIM)],
                rows.at[sl], sems).wait()
        pltpu.make_async_copy(
            table_hbm.at[pl.ds(0, DIM), pl.ds(0, WIN)],
            wins[r], semw[r]).wait()
        wbase = j * WIN

        def one_table(bv, vals, poss, off, dst_hbm, tsl):
            e16 = bv[pl.ds(j, LANES)]
            s = e16[0]
            e = e16[1]
            for g in range(NG):
                sl = tsl * NG + g
                posb[sl, :] = jnp.full((LANES,), DUMP + sl, jnp.int32)
                i = s + g * LANES

                @pl.when(i < e)
                def _():
                    li = i - off
                    sv16 = vals[pl.ds(li, LANES)]
                    sp16 = poss[pl.ds(li, LANES)]
                    msk = (lanes + i) < e
                    cols = jnp.clip(sv16 - wbase, 0, WIN - 1)
                    posb[sl, :] = jnp.where(msk, sp16, DUMP + sl)
                    for d in range(DIM):
                        gv = plsc.load_gather(
                            wins[r], [jnp.full((LANES,), d, jnp.int32), cols])
                        plsc.store_scatter(
                            rows.at[sl],
                            [lanes, jnp.full((LANES,), d, jnp.int32)], gv)

                pltpu.async_copy(rows.at[sl], dst_hbm.at[posb.at[sl]], sems)

        one_table(btv, sval, pval, toff, u2_hbm, 0)
        one_table(bcv, cval, qval, coff, v2_hbm, 1)
        fire_load(j + RING, r)

    nsuper = (WPW_LO + 1 + RING - 1) // RING

    def super_iter(q, carry):
        for r in range(RING):
            j = j0 + q * RING + r

            @pl.when(j < j0 + nwin)
            def _():
                do_window(j, r)
        return carry

    lax.fori_loop(0, nsuper, super_iter, 0)

    for sl in range(2 * NG):
        pltpu.make_async_copy(
            u2_hbm.at[pl.ds(0, LANES), pl.ds(0, 2 * DIM)],
            rows.at[sl], sems).wait()


def _dot_body(t_hbm, c_hbm, u2_hbm, v2_hbm, aux_hbm, out_hbm,
              tv, cv, ub, vb, auxv, out_v, sem):
    wid = lax.axis_index("s") * NUM_CORES + lax.axis_index("c")
    base = wid * BPW

    pltpu.sync_copy(t_hbm.at[pl.ds(base, BPW)], tv)
    pltpu.sync_copy(c_hbm.at[pl.ds(base, BPW)], cv)
    pltpu.sync_copy(aux_hbm, auxv)

    lanes = lax.iota(jnp.int32, LANES)
    NCH = BPW // 128

    for ch in range(NCH):
        row0 = pl.multiple_of(base + ch * 128, 8)
        pltpu.sync_copy(u2_hbm.at[pl.ds(row0, 128), pl.ds(0, 2 * DIM)], ub)
        pltpu.sync_copy(v2_hbm.at[pl.ds(row0, 128), pl.ds(0, 2 * DIM)], vb)

        def group(g, carry):
            i0 = ch * 128 + g * LANES
            t16 = tv[pl.ds(i0, LANES)]
            c16 = cv[pl.ds(i0, LANES)]
            tm = t16 >= VTAIL
            cm = c16 >= VTAIL
            ta = jnp.clip(t16 - VTAIL, 0, DIM - 1)
            ca = jnp.clip(c16 - VTAIL, 0, DIM - 1)
            rloc = lanes + g * LANES
            acc = jnp.zeros((LANES,), jnp.float32)
            for d in range(DIM):
                dsp = jnp.full((LANES,), d, jnp.int32)
                uu = plsc.load_gather(ub, [rloc, dsp])
                vv = plsc.load_gather(vb, [rloc, dsp])
                au = plsc.load_gather(auxv, [ta, dsp])
                av = plsc.load_gather(auxv, [ca, dsp])
                u = jnp.where(tm, au, uu)
                v = jnp.where(cm, av, vv)
                acc = acc + u * v
            out_v[pl.ds(i0, LANES)] = acc
            return carry

        lax.fori_loop(0, 8, group, 0)

    pltpu.sync_copy(out_v, out_hbm.at[pl.ds(base, BPW)])


@jax.jit
def _skipgram(t32, c32, table_t, aux):
    pt = jnp.argsort(t32).astype(jnp.int32)
    st = t32[pt]
    pc = jnp.argsort(c32).astype(jnp.int32)
    sc_ = c32[pc]
    edges = jnp.minimum(jnp.arange(NEDGE, dtype=jnp.int32) * WIN, VOCAB)
    bt = jnp.searchsorted(st, edges, method="sort").astype(jnp.int32)
    bc = jnp.searchsorted(sc_, edges, method="sort").astype(jnp.int32)

    mesh = plsc.VectorSubcoreMesh(core_axis_name="c", subcore_axis_name="s")
    u2, v2 = pl.kernel(
        _sweep_body,
        out_type=(jax.ShapeDtypeStruct((IROWS, 2 * DIM), jnp.float32),
                  jax.ShapeDtypeStruct((IROWS, 2 * DIM), jnp.float32)),
        mesh=mesh,
        scratch_types=[
            pltpu.VMEM((NEDGE,), jnp.int32),
            pltpu.VMEM((NEDGE,), jnp.int32),
            pltpu.VMEM((CAP,), jnp.int32),
            pltpu.VMEM((CAP,), jnp.int32),
            pltpu.VMEM((CAP,), jnp.int32),
            pltpu.VMEM((CAP,), jnp.int32),
            pltpu.VMEM((DIM, WIN), jnp.float32),
            pltpu.VMEM((DIM, WIN), jnp.float32),
            pltpu.VMEM((2 * NG, LANES, 2 * DIM), jnp.float32),
            pltpu.VMEM((2 * NG, LANES), jnp.int32),
            pltpu.SemaphoreType.DMA,
            pltpu.SemaphoreType.DMA,
            pltpu.SemaphoreType.DMA,
        ],
        compiler_params=pltpu.CompilerParams(needs_layout_passes=False),
    )(st, pt, sc_, pc, bt, bc, table_t)

    return pl.kernel(
        _dot_body,
        out_type=jax.ShapeDtypeStruct((B,), jnp.float32),
        mesh=mesh,
        scratch_types=[
            pltpu.VMEM((BPW,), jnp.int32),
            pltpu.VMEM((BPW,), jnp.int32),
            pltpu.VMEM((128, 2 * DIM), jnp.float32),
            pltpu.VMEM((128, 2 * DIM), jnp.float32),
            pltpu.VMEM((DIM, 2 * DIM), jnp.float32),
            pltpu.VMEM((BPW,), jnp.float32),
            pltpu.SemaphoreType.DMA,
        ],
        compiler_params=pltpu.CompilerParams(needs_layout_passes=False),
    )(t32, c32, u2, v2, aux)


def kernel(target, context, embedding_weights):
    t32 = target.astype(jnp.int32)
    c32 = context.astype(jnp.int32)
    aux = jnp.pad(embedding_weights[VTAIL:], ((0, 0), (0, DIM)))
    return _skipgram(t32, c32, embedding_weights.T, aux)


# speedup vs baseline: 5.4985x; 3.2204x over previous
import jax
import jax.numpy as jnp
from jax import lax
from jax.experimental import pallas as pl
from jax.experimental.pallas import tpu as pltpu
from jax.experimental.pallas import tpu_sc as plsc

VOCAB = 1000000
DIM = 64
B = 16384

NUM_CORES = 2
NUM_SUBCORES = 16
LANES = 16
NW = NUM_CORES * NUM_SUBCORES
BPW = B // NW
CHUNK = 128
NCHUNK = BPW // CHUNK
WIDE = 2 * DIM


def _sc_body(t_hbm, c_hbm, table_hbm, out_hbm,
             idx_t, idx_c, u0, u1, v0, v1, out_v,
             sem0, sem1):
    wid = lax.axis_index("s") * NUM_CORES + lax.axis_index("c")
    base = wid * BPW

    pltpu.sync_copy(t_hbm.at[pl.ds(base, BPW)], idx_t)
    pltpu.sync_copy(c_hbm.at[pl.ds(base, BPW)], idx_c)

    ubufs, vbufs, sems = [u0, u1], [v0, v1], [sem0, sem1]

    def fire(j):
        k = j % 2
        sl = pl.ds(j * CHUNK, CHUNK)
        cu = pltpu.async_copy(table_hbm.at[idx_t.at[sl]], ubufs[k], sems[k])
        cv = pltpu.async_copy(table_hbm.at[idx_c.at[sl]], vbufs[k], sems[k])
        return cu, cv

    inflight = fire(0)

    for j in range(NCHUNK):
        cu, cv = inflight
        if j + 1 < NCHUNK:
            nxt = fire(j + 1)
        cu.wait()
        cv.wait()
        ubuf, vbuf = ubufs[j % 2], vbufs[j % 2]

        def group(g, carry):
            rows16 = lax.iota(jnp.int32, 16) + g * LANES
            sl16 = pl.ds(j * CHUNK + g * LANES, LANES)
            acc = jnp.zeros((LANES,), jnp.float32)
            for d in range(DIM):
                dsp = jnp.full((LANES,), d, jnp.int32)
                u = plsc.load_gather(ubuf, [rows16, dsp])
                v = plsc.load_gather(vbuf, [rows16, dsp])
                acc = acc + u * v
            out_v[sl16] = acc
            return carry

        lax.fori_loop(0, CHUNK // LANES, group, 0)
        if j + 1 < NCHUNK:
            inflight = nxt

    pltpu.sync_copy(out_v, out_hbm.at[pl.ds(base, BPW)])


@jax.jit
def _skipgram(t32, c32, table2):
    mesh = plsc.VectorSubcoreMesh(core_axis_name="c", subcore_axis_name="s")
    return pl.kernel(
        _sc_body,
        out_type=jax.ShapeDtypeStruct((B,), jnp.float32),
        mesh=mesh,
        scratch_types=[
            pltpu.VMEM((BPW,), jnp.int32),
            pltpu.VMEM((BPW,), jnp.int32),
            pltpu.VMEM((CHUNK, WIDE), jnp.float32),
            pltpu.VMEM((CHUNK, WIDE), jnp.float32),
            pltpu.VMEM((CHUNK, WIDE), jnp.float32),
            pltpu.VMEM((CHUNK, WIDE), jnp.float32),
            pltpu.VMEM((BPW,), jnp.float32),
            pltpu.SemaphoreType.DMA,
            pltpu.SemaphoreType.DMA,
        ],
        compiler_params=pltpu.CompilerParams(needs_layout_passes=False),
    )(t32, c32, table2)


def kernel(target, context, embedding_weights):
    table2 = jnp.pad(embedding_weights, ((0, 0), (0, DIM)))
    return _skipgram(target.astype(jnp.int32), context.astype(jnp.int32),
                     table2)
